# baseline (speedup 1.0000x reference)
import functools

import jax
import jax.numpy as jnp
from jax import lax
from jax.experimental import pallas as pl
from jax.experimental.pallas import tpu as pltpu
from jax.experimental.pallas import tpu_sc as plsc

NUM_CORES = 2
NUM_SUBCORES = 16
NW = NUM_CORES * NUM_SUBCORES

C = 96


def _make_sc_interp(npix: int, nfreq: int, nsrc: int):
    assert nsrc % (NW * C) == 0
    src_per_w = nsrc // NW
    nch = src_per_w // C
    assert nch % 2 == 0
    c4 = 4 * C
    mesh = plsc.VectorSubcoreMesh(core_axis_name="c", subcore_axis_name="s")

    @functools.partial(
        pl.kernel,
        mesh=mesh,
        compiler_params=pltpu.CompilerParams(
            use_tc_tiling_on_sc=False, needs_layout_passes=False),
        out_type=jax.ShapeDtypeStruct((nfreq, nsrc), jnp.float32),
        scratch_types=[
            pltpu.VMEM((nch, c4), jnp.int32),
            pltpu.VMEM((nch, 4, C), jnp.float32),
            pltpu.VMEM((c4, nfreq), jnp.float32),
            pltpu.VMEM((c4, nfreq), jnp.float32),
            pltpu.VMEM((nfreq, C), jnp.float32),
            pltpu.VMEM((nfreq, C), jnp.float32),
            pltpu.SemaphoreType.DMA,
            pltpu.SemaphoreType.DMA,
            pltpu.SemaphoreType.DMA,
            pltpu.SemaphoreType.DMA,
        ],
    )
    def sc_interp(table, idxp, wp, out, idx_all, w_all,
                  rows0, rows1, ov0, ov1, sg0, sg1, so0, so1):
        wid = lax.axis_index("s") * NUM_CORES + lax.axis_index("c")
        base_w = wid * src_per_w
        iota16 = lax.iota(jnp.int32, 16)

        pltpu.sync_copy(idxp.at[wid], idx_all)
        pltpu.sync_copy(wp.at[wid], w_all)

        def gstart(c, rows, sem):
            pltpu.async_copy(table.at[idx_all.at[c]], rows, sem)

        def gwait(rows, sem):
            pltpu.make_async_copy(table.at[idx_all.at[0]], rows, sem).wait()

        def ostart(c, ov, sem):
            pltpu.async_copy(ov, out.at[:, pl.ds(base_w + c * C, C)], sem)

        def owait(ov, sem):
            pltpu.make_async_copy(ov, out.at[:, pl.ds(base_w, C)], sem).wait()

        def compute(c, rows, ov):
            for c16 in range(C // 16):
                lanes = iota16 + (c16 * 16)
                ridx = [lanes + k * C for k in range(4)]
                ws = [w_all[c, k, pl.ds(c16 * 16, 16)] for k in range(4)]

                def fgbody(fg, carry, ridx=ridx, ws=ws, c16=c16):
                    f0 = fg * 16
                    for j in range(16):
                        colf = jnp.full((16,), f0 + j, dtype=jnp.int32)
                        a0 = ws[0] * plsc.load_gather(rows, [ridx[0], colf])
                        a1 = ws[1] * plsc.load_gather(rows, [ridx[1], colf])
                        a2 = ws[2] * plsc.load_gather(rows, [ridx[2], colf])
                        a3 = ws[3] * plsc.load_gather(rows, [ridx[3], colf])
                        ov[f0 + j, pl.ds(c16 * 16, 16)] = jnp.abs((a0 + a1) + (a2 + a3))
                    return carry

                lax.fori_loop(0, nfreq // 16, fgbody, 0)

        gstart(0, rows0, sg0)

        def pair(i, carry):
            c0 = 2 * i
            c1 = c0 + 1
            gstart(c1, rows1, sg1)
            gwait(rows0, sg0)

            @pl.when(i > 0)
            def _():
                owait(ov0, so0)

            compute(c0, rows0, ov0)
            ostart(c0, ov0, so0)

            @pl.when(c0 + 2 < nch)
            def _():
                gstart(c0 + 2, rows0, sg0)

            gwait(rows1, sg1)

            @pl.when(i > 0)
            def _():
                owait(ov1, so1)

            compute(c1, rows1, ov1)
            ostart(c1, ov1, so1)
            return carry

        lax.fori_loop(0, nch // 2, pair, 0)
        owait(ov0, so0)
        owait(ov1, so1)

    return sc_interp


def kernel(params, inds, wgts):
    npol, npol2, nmodel, nfreq, npix = params.shape
    nnbr, nsrc = inds.shape
    src_per_w = nsrc // NW
    nch = src_per_w // C

    table = jnp.transpose(params.reshape(nfreq, npix))
    idxp = (inds.reshape(nnbr, NW, nch, C)
            .transpose(1, 2, 0, 3).reshape(NW, nch, nnbr * C))
    wp = wgts.reshape(nnbr, NW, nch, C).transpose(1, 2, 0, 3)

    out = _make_sc_interp(npix, nfreq, nsrc)(table, idxp, wp)
    return out.reshape(npol, npol2, nmodel, nfreq, nsrc)


# speedup vs baseline: 2.1105x; 2.1105x over previous
import functools

import jax
import jax.numpy as jnp
from jax import lax
from jax.experimental import pallas as pl
from jax.experimental.pallas import tpu as pltpu
from jax.experimental.pallas import tpu_sc as plsc

NUM_CORES = 2
NUM_SUBCORES = 16
NW = NUM_CORES * NUM_SUBCORES

C = 96


def _make_sc_interp(npix: int, nfreq: int, nsrc: int):
    assert nsrc % (NW * C) == 0
    src_per_w = nsrc // NW
    nch = src_per_w // C
    assert nch % 2 == 0
    c4 = 4 * C
    mesh = plsc.VectorSubcoreMesh(core_axis_name="c", subcore_axis_name="s")

    @functools.partial(
        pl.kernel,
        mesh=mesh,
        compiler_params=pltpu.CompilerParams(
            use_tc_tiling_on_sc=False, needs_layout_passes=False),
        out_type=jax.ShapeDtypeStruct((nfreq, nsrc), jnp.float32),
        scratch_types=[
            pltpu.VMEM((nch, c4), jnp.int32),
            pltpu.VMEM((nch, 4, C), jnp.float32),
            pltpu.VMEM((c4, nfreq), jnp.float32),
            pltpu.VMEM((c4, nfreq), jnp.float32),
            pltpu.VMEM((nfreq, C), jnp.float32),
            pltpu.VMEM((nfreq, C), jnp.float32),
            pltpu.SemaphoreType.DMA,
            pltpu.SemaphoreType.DMA,
            pltpu.SemaphoreType.DMA,
            pltpu.SemaphoreType.DMA,
        ],
    )
    def sc_interp(table, idxp, wp, out, idx_all, w_all,
                  rows0, rows1, ov0, ov1, sg0, sg1, so0, so1):
        wid = lax.axis_index("s") * NUM_CORES + lax.axis_index("c")
        base_w = wid * src_per_w
        iota16 = lax.iota(jnp.int32, 16)

        pltpu.sync_copy(idxp.at[wid], idx_all)
        pltpu.sync_copy(wp.at[wid], w_all)

        def gstart(c, rows, sem):
            pltpu.async_copy(table.at[idx_all.at[c]], rows, sem)

        def gwait(rows, sem):
            pltpu.make_async_copy(table.at[idx_all.at[0]], rows, sem).wait()

        def ostart(c, ov, sem):
            pltpu.async_copy(ov, out.at[:, pl.ds(base_w + c * C, C)], sem)

        def owait(ov, sem):
            pltpu.make_async_copy(ov, out.at[:, pl.ds(base_w, C)], sem).wait()

        def compute(c, rows, ov):
            for c16 in range(C // 16):
                lanes = iota16 + (c16 * 16)
                ridx = [lanes + k * C for k in range(4)]
                ws = [w_all[c, k, pl.ds(c16 * 16, 16)] for k in range(4)]

                def fgbody(fg, carry, ridx=ridx, ws=ws, lanes=lanes):
                    base = iota16 + fg * 4
                    for j in range(4):
                        rot = (base + j) & (nfreq - 1)
                        a0 = ws[0] * plsc.load_gather(rows, [ridx[0], rot])
                        a1 = ws[1] * plsc.load_gather(rows, [ridx[1], rot])
                        a2 = ws[2] * plsc.load_gather(rows, [ridx[2], rot])
                        a3 = ws[3] * plsc.load_gather(rows, [ridx[3], rot])
                        val = jnp.abs((a0 + a1) + (a2 + a3))
                        plsc.store_scatter(ov, [rot, lanes], val)
                    return carry

                lax.fori_loop(0, nfreq // 4, fgbody, 0)

        gstart(0, rows0, sg0)

        def pair(i, carry):
            c0 = 2 * i
            c1 = c0 + 1
            gstart(c1, rows1, sg1)
            gwait(rows0, sg0)

            @pl.when(i > 0)
            def _():
                owait(ov0, so0)

            compute(c0, rows0, ov0)
            ostart(c0, ov0, so0)

            @pl.when(c0 + 2 < nch)
            def _():
                gstart(c0 + 2, rows0, sg0)

            gwait(rows1, sg1)

            @pl.when(i > 0)
            def _():
                owait(ov1, so1)

            compute(c1, rows1, ov1)
            ostart(c1, ov1, so1)
            return carry

        lax.fori_loop(0, nch // 2, pair, 0)
        owait(ov0, so0)
        owait(ov1, so1)

    return sc_interp


def kernel(params, inds, wgts):
    npol, npol2, nmodel, nfreq, npix = params.shape
    nnbr, nsrc = inds.shape
    src_per_w = nsrc // NW
    nch = src_per_w // C

    table = jnp.transpose(params.reshape(nfreq, npix))
    idxp = (inds.reshape(nnbr, NW, nch, C)
            .transpose(1, 2, 0, 3).reshape(NW, nch, nnbr * C))
    wp = wgts.reshape(nnbr, NW, nch, C).transpose(1, 2, 0, 3)

    out = _make_sc_interp(npix, nfreq, nsrc)(table, idxp, wp)
    return out.reshape(npol, npol2, nmodel, nfreq, nsrc)
